# baseline (speedup 1.0000x reference)
import functools

import jax
import jax.numpy as jnp
from jax import lax
from jax.experimental import pallas as pl
from jax.experimental.pallas import tpu as pltpu
from jax.experimental.pallas import tpu_sc as plsc

NC = 2
NS = 16
K = 128
NBUF = 4


def _make_agg_kernel(n_rows, dh, j_chunks):
    mesh = plsc.VectorSubcoreMesh(
        core_axis_name="c", subcore_axis_name="s", num_cores=NC,
        num_subcores=NS)
    rows_per_tile = n_rows // NS
    init_chunks = rows_per_tile // K

    @functools.partial(
        pl.kernel,
        out_type=jax.ShapeDtypeStruct((NC, n_rows, dh), jnp.float32),
        mesh=mesh,
        scratch_types=[
            pltpu.VMEM((j_chunks, K), jnp.int32),
            pltpu.VMEM((j_chunks, K), jnp.int32),
            [pltpu.VMEM((K, dh), jnp.float32) for _ in range(NBUF)],
            pltpu.VMEM_SHARED((n_rows, dh), jnp.float32),
            [pltpu.SemaphoreType.DMA for _ in range(NBUF + 2)],
        ],
        compiler_params=pltpu.CompilerParams(use_tc_tiling_on_sc=False),
    )
    def agg(h_hbm, src_hbm, dst_hbm, out_hbm, src_v, dst_v, rows, acc, sems):
        cid = lax.axis_index("c")
        sid = lax.axis_index("s")
        ci = pltpu.async_copy(src_hbm.at[sid], src_v, sems[NBUF])
        cj = pltpu.async_copy(dst_hbm.at[sid], dst_v, sems[NBUF + 1])
        base = sid * rows_per_tile
        for i in range(init_chunks):
            pltpu.sync_copy(h_hbm.at[cid, pl.ds(base + i * K, K)],
                            acc.at[pl.ds(base + i * K, K)])
        ci.wait()
        cj.wait()
        plsc.subcore_barrier()

        def gather(j, b):
            return pltpu.async_copy(h_hbm.at[cid].at[src_v.at[j]], rows[b],
                                    sems[b])

        def drain_and_scatter(j, b):
            pltpu.make_async_copy(h_hbm.at[cid].at[src_v.at[j]], rows[b],
                                  sems[b]).wait()
            pltpu.sync_copy(rows[b], acc.at[dst_v.at[j]], add=True)

        for b in range(NBUF):
            gather(b, b)

        @pl.loop(0, j_chunks // NBUF - 1)
        def _(g):
            for b in range(NBUF):
                j = g * NBUF + b
                drain_and_scatter(j, b)
                gather(j + NBUF, b)

        for b in range(NBUF):
            drain_and_scatter(j_chunks - NBUF + b, b)

        plsc.subcore_barrier()
        for i in range(init_chunks):
            pltpu.sync_copy(acc.at[pl.ds(base + i * K, K)],
                            out_hbm.at[cid, pl.ds(base + i * K, K)])

    return agg


def _mlp_call(a, Ws, bs, n_out_rows, blk, split_out):
    dh = a.shape[-1]
    d_out = Ws[2].shape[1]

    def body(a_r, w0, b0, w1, b1, w2, b2, o_r):
        h = jnp.concatenate([a_r[0], a_r[1]], axis=1)
        h = jnp.maximum(
            jnp.dot(h, w0[...], preferred_element_type=jnp.float32) + b0[...],
            0.0)
        h = jnp.maximum(
            jnp.dot(h, w1[...], preferred_element_type=jnp.float32) + b1[...],
            0.0)
        h = jnp.dot(h, w2[...], preferred_element_type=jnp.float32) + b2[...]
        if split_out:
            o_r[0] = h[:, :dh]
            o_r[1] = h[:, dh:]
        else:
            o_r[...] = h

    full = lambda w: pl.BlockSpec(w.shape, lambda i: (0, 0))
    if split_out:
        out_spec = pl.BlockSpec((NC, blk, dh), lambda i: (0, i, 0))
        out_shape = jax.ShapeDtypeStruct((NC, n_out_rows, dh), jnp.float32)
    else:
        out_spec = pl.BlockSpec((blk, d_out), lambda i: (i, 0))
        out_shape = jax.ShapeDtypeStruct((n_out_rows, d_out), jnp.float32)
    return pl.pallas_call(
        body,
        grid=(n_out_rows // blk,),
        in_specs=[
            pl.BlockSpec((NC, blk, dh), lambda i: (0, i, 0)),
            full(Ws[0]), full(bs[0]), full(Ws[1]), full(bs[1]),
            full(Ws[2]), full(bs[2]),
        ],
        out_specs=out_spec,
        out_shape=out_shape,
    )(a, Ws[0], bs[0], Ws[1], bs[1], Ws[2], bs[2])


def kernel(x, edge_index, params):
    n, d = x.shape
    dh = d // NC
    e = edge_index.shape[1]
    n_rows = ((n + NS * K - 1) // (NS * K)) * NS * K
    j_chunks = -(-(-(-e // (NS * K))) // NBUF) * NBUF
    j_chunks = max(j_chunks, 2 * NBUF)
    slots_pt = j_chunks * K
    pad = NS * slots_pt - e

    src = edge_index[0].astype(jnp.int32)
    dst = edge_index[1].astype(jnp.int32)
    pad_src = jnp.arange(pad, dtype=jnp.int32) % n
    pad_dst = n + jnp.arange(pad, dtype=jnp.int32) % (n_rows - n)
    src_w = jnp.concatenate([src, pad_src]).reshape(NS, j_chunks, K)
    dst_w = jnp.concatenate([dst, pad_dst]).reshape(NS, j_chunks, K)

    x_pad = jnp.zeros((n_rows, d), jnp.float32).at[:n].set(x)
    h = x_pad.reshape(n_rows, NC, dh).transpose(1, 0, 2)

    agg = _make_agg_kernel(n_rows, dh, j_chunks)

    for li, (Ws, bs) in enumerate(params):
        a = agg(h, src_w, dst_w)
        bs2 = tuple(b.reshape(1, -1) for b in bs)
        last = li == len(params) - 1
        if last:
            h = _mlp_call(a, Ws, bs2, n, 2000, split_out=False)
        else:
            h = _mlp_call(a, Ws, bs2, n_rows, 1024, split_out=True)
    return h


# speedup vs baseline: 1.2590x; 1.2590x over previous
import functools

import jax
import jax.numpy as jnp
from jax import lax
from jax.experimental import pallas as pl
from jax.experimental.pallas import tpu as pltpu
from jax.experimental.pallas import tpu_sc as plsc

NC = 2
NS = 16
K = 128
NBUF = 4


def _make_agg_kernel(n_rows, dh, j_chunks):
    mesh = plsc.VectorSubcoreMesh(
        core_axis_name="c", subcore_axis_name="s", num_cores=NC,
        num_subcores=NS)
    rows_per_tile = n_rows // NS
    init_chunks = rows_per_tile // K

    @functools.partial(
        pl.kernel,
        out_type=jax.ShapeDtypeStruct((NC, n_rows, dh), jnp.float32),
        mesh=mesh,
        scratch_types=[
            pltpu.VMEM((j_chunks, K), jnp.int32),
            pltpu.VMEM((j_chunks, K), jnp.int32),
            [pltpu.VMEM((K, dh), jnp.float32) for _ in range(NBUF)],
            pltpu.VMEM_SHARED((n_rows, dh), jnp.float32),
            [pltpu.SemaphoreType.DMA for _ in range(NBUF + 2)],
        ],
        compiler_params=pltpu.CompilerParams(use_tc_tiling_on_sc=False),
    )
    def agg(h_hbm, src_hbm, dst_hbm, out_hbm, src_v, dst_v, rows, acc, sems):
        cid = lax.axis_index("c")
        sid = lax.axis_index("s")
        ci = pltpu.async_copy(src_hbm.at[sid], src_v, sems[NBUF])
        cj = pltpu.async_copy(dst_hbm.at[sid], dst_v, sems[NBUF + 1])
        base = sid * rows_per_tile
        for i in range(init_chunks):
            pltpu.sync_copy(h_hbm.at[cid, pl.ds(base + i * K, K)],
                            acc.at[pl.ds(base + i * K, K)])
        ci.wait()
        cj.wait()
        plsc.subcore_barrier()

        def gather(j, b):
            return pltpu.async_copy(h_hbm.at[cid].at[src_v.at[j]], rows[b],
                                    sems[b])

        def drain_and_scatter(j, b):
            pltpu.make_async_copy(h_hbm.at[cid].at[src_v.at[j]], rows[b],
                                  sems[b]).wait()
            pltpu.sync_copy(rows[b], acc.at[dst_v.at[j]], add=True)

        for b in range(NBUF):
            gather(b, b)

        @pl.loop(0, j_chunks // NBUF - 1)
        def _(g):
            for b in range(NBUF):
                j = g * NBUF + b
                drain_and_scatter(j, b)
                gather(j + NBUF, b)

        for b in range(NBUF):
            drain_and_scatter(j_chunks - NBUF + b, b)

        plsc.subcore_barrier()
        for i in range(init_chunks):
            pltpu.sync_copy(acc.at[pl.ds(base + i * K, K)],
                            out_hbm.at[cid, pl.ds(base + i * K, K)])

    return agg


def _mlp_call(a_pairs, Ws, bs, blk2, n_out_pairs, split_out):
    dh = 64
    d_out = Ws[2].shape[1]

    def body(a_r, w0, b0, w1, b1, w2, b2, o_r):
        a0 = a_r[0]
        a1 = a_r[1]

        def mlp(h):
            h = jnp.maximum(
                jnp.dot(h, w0[...], preferred_element_type=jnp.float32)
                + b0[...], 0.0)
            h = jnp.maximum(
                jnp.dot(h, w1[...], preferred_element_type=jnp.float32)
                + b1[...], 0.0)
            return (jnp.dot(h, w2[...], preferred_element_type=jnp.float32)
                    + b2[...])

        re = mlp(jnp.concatenate([a0[:, :dh], a1[:, :dh]], axis=1))
        ro = mlp(jnp.concatenate([a0[:, dh:], a1[:, dh:]], axis=1))
        if split_out:
            o_r[0] = jnp.concatenate([re[:, :dh], ro[:, :dh]], axis=1)
            o_r[1] = jnp.concatenate([re[:, dh:], ro[:, dh:]], axis=1)
        else:
            o_r[...] = jnp.stack([re, ro], axis=1).reshape(2 * blk2, d_out)

    full = lambda w: pl.BlockSpec(w.shape, lambda i: (0, 0))
    if split_out:
        out_spec = pl.BlockSpec((NC, blk2, 128), lambda i: (0, i, 0))
        out_shape = jax.ShapeDtypeStruct((NC, n_out_pairs, 128), jnp.float32)
    else:
        out_spec = pl.BlockSpec((2 * blk2, d_out), lambda i: (i, 0))
        out_shape = jax.ShapeDtypeStruct((2 * n_out_pairs, d_out), jnp.float32)
    return pl.pallas_call(
        body,
        grid=(n_out_pairs // blk2,),
        in_specs=[
            pl.BlockSpec((NC, blk2, 128), lambda i: (0, i, 0)),
            full(Ws[0]), full(bs[0]), full(Ws[1]), full(bs[1]),
            full(Ws[2]), full(bs[2]),
        ],
        out_specs=out_spec,
        out_shape=out_shape,
    )(a_pairs, Ws[0], bs[0], Ws[1], bs[1], Ws[2], bs[2])


def kernel(x, edge_index, params):
    n, d = x.shape
    dh = d // NC
    e = edge_index.shape[1]
    n_rows = ((n + NS * K - 1) // (NS * K)) * NS * K
    j_chunks = -(-(-(-e // (NS * K))) // NBUF) * NBUF
    j_chunks = max(j_chunks, 2 * NBUF)
    slots_pt = j_chunks * K
    pad = NS * slots_pt - e

    src = edge_index[0].astype(jnp.int32)
    dst = edge_index[1].astype(jnp.int32)
    pad_src = jnp.arange(pad, dtype=jnp.int32) % n
    pad_dst = n + jnp.arange(pad, dtype=jnp.int32) % (n_rows - n)
    src_w = jnp.concatenate([src, pad_src]).reshape(NS, j_chunks, K)
    dst_w = jnp.concatenate([dst, pad_dst]).reshape(NS, j_chunks, K)

    x_pad = jnp.zeros((n_rows, d), jnp.float32).at[:n].set(x)
    h = x_pad.reshape(n_rows, NC, dh).transpose(1, 0, 2)

    agg = _make_agg_kernel(n_rows, dh, j_chunks)

    for li, (Ws, bs) in enumerate(params):
        a = agg(h, src_w, dst_w)
        a_pairs = a.reshape(NC, n_rows // 2, d)
        bs2 = tuple(b.reshape(1, -1) for b in bs)
        last = li == len(params) - 1
        if last:
            h = _mlp_call(a_pairs, Ws, bs2, n // 2 // 5, n // 2,
                          split_out=False)
        else:
            hp = _mlp_call(a_pairs, Ws, bs2, 1024, n_rows // 2,
                           split_out=True)
            h = hp.reshape(NC, n_rows, dh)
    return h


# speedup vs baseline: 1.3174x; 1.0464x over previous
import functools

import jax
import jax.numpy as jnp
from jax import lax
from jax.experimental import pallas as pl
from jax.experimental.pallas import tpu as pltpu
from jax.experimental.pallas import tpu_sc as plsc

NC = 2
NS = 16
K = 128
NBUF = 4


def _make_agg_kernel(n_rows, dh, n_chunks):
    mesh = plsc.VectorSubcoreMesh(
        core_axis_name="c", subcore_axis_name="s", num_cores=NC,
        num_subcores=NS)
    rows_per_tile = n_rows // NS
    init_chunks = rows_per_tile // K
    cpt = -(-n_chunks // NS)
    cpt_pad = -(-cpt // NBUF) * NBUF

    @functools.partial(
        pl.kernel,
        out_type=jax.ShapeDtypeStruct((NC, n_rows, dh), jnp.float32),
        mesh=mesh,
        scratch_types=[
            pltpu.VMEM((cpt, K), jnp.int32),
            pltpu.VMEM((cpt, K), jnp.int32),
            [pltpu.VMEM((K, dh), jnp.float32) for _ in range(NBUF)],
            pltpu.VMEM_SHARED((n_rows, dh), jnp.float32),
            [pltpu.SemaphoreType.DMA for _ in range(NBUF + 2)],
        ],
        compiler_params=pltpu.CompilerParams(use_tc_tiling_on_sc=False),
    )
    def agg(h_hbm, ei_hbm, out_hbm, src_v, dst_v, rows, acc, sems):
        cid = lax.axis_index("c")
        sid = lax.axis_index("s")
        n_full = 0
        copies = []
        for t in range(NS):
            mc = min(cpt, max(0, n_chunks - t * cpt))
            if mc == cpt:
                n_full = t + 1
            elif mc > 0:
                copies.append((t, mc))
        if n_full:
            @pl.when(sid < n_full)
            def _():
                pltpu.async_copy(ei_hbm.at[0, pl.ds(sid * cpt, cpt)], src_v,
                                 sems[NBUF])
                pltpu.async_copy(ei_hbm.at[1, pl.ds(sid * cpt, cpt)], dst_v,
                                 sems[NBUF + 1])
        for (t, mc) in copies:
            @pl.when(sid == t)
            def _(t=t, mc=mc):
                pltpu.async_copy(ei_hbm.at[0, pl.ds(t * cpt, mc)],
                                 src_v.at[pl.ds(0, mc)], sems[NBUF])
                pltpu.async_copy(ei_hbm.at[1, pl.ds(t * cpt, mc)],
                                 dst_v.at[pl.ds(0, mc)], sems[NBUF + 1])
        my_chunks = jnp.minimum(jnp.maximum(n_chunks - sid * cpt, 0), cpt)
        base = sid * rows_per_tile
        for i in range(init_chunks):
            pltpu.sync_copy(h_hbm.at[cid, pl.ds(base + i * K, K)],
                            acc.at[pl.ds(base + i * K, K)])
        if n_full:
            @pl.when(sid < n_full)
            def _():
                pltpu.make_async_copy(ei_hbm.at[0, pl.ds(0, cpt)], src_v,
                                      sems[NBUF]).wait()
                pltpu.make_async_copy(ei_hbm.at[1, pl.ds(0, cpt)], dst_v,
                                      sems[NBUF + 1]).wait()
        for (t, mc) in copies:
            @pl.when(sid == t)
            def _(t=t, mc=mc):
                pltpu.make_async_copy(ei_hbm.at[0, pl.ds(0, mc)],
                                      src_v.at[pl.ds(0, mc)],
                                      sems[NBUF]).wait()
                pltpu.make_async_copy(ei_hbm.at[1, pl.ds(0, mc)],
                                      dst_v.at[pl.ds(0, mc)],
                                      sems[NBUF + 1]).wait()
        plsc.subcore_barrier()

        def gather(j, b):
            @pl.when(j < my_chunks)
            def _():
                pltpu.async_copy(h_hbm.at[cid].at[src_v.at[j]], rows[b],
                                 sems[b])

        def drain_and_scatter(j, b):
            @pl.when(j < my_chunks)
            def _():
                pltpu.make_async_copy(h_hbm.at[cid].at[src_v.at[j]], rows[b],
                                      sems[b]).wait()
                pltpu.sync_copy(rows[b], acc.at[dst_v.at[j]], add=True)

        for b in range(NBUF):
            gather(b, b)

        @pl.loop(0, cpt_pad // NBUF - 1)
        def _(g):
            for b in range(NBUF):
                j = g * NBUF + b
                drain_and_scatter(j, b)
                gather(j + NBUF, b)

        for b in range(NBUF):
            drain_and_scatter(cpt_pad - NBUF + b, b)

        plsc.subcore_barrier()
        for i in range(init_chunks):
            pltpu.sync_copy(acc.at[pl.ds(base + i * K, K)],
                            out_hbm.at[cid, pl.ds(base + i * K, K)])

    return agg


def _mlp_call(a_pairs, Ws, bs, blk2, n_out_pairs, split_out):
    dh = 64
    d_out = Ws[2].shape[1]

    def body(a_r, w0, b0, w1, b1, w2, b2, o_r):
        a0 = a_r[0]
        a1 = a_r[1]

        def mlp(h):
            h = jnp.maximum(
                jnp.dot(h, w0[...], preferred_element_type=jnp.float32)
                + b0[...], 0.0)
            h = jnp.maximum(
                jnp.dot(h, w1[...], preferred_element_type=jnp.float32)
                + b1[...], 0.0)
            return (jnp.dot(h, w2[...], preferred_element_type=jnp.float32)
                    + b2[...])

        re = mlp(jnp.concatenate([a0[:, :dh], a1[:, :dh]], axis=1))
        ro = mlp(jnp.concatenate([a0[:, dh:], a1[:, dh:]], axis=1))
        if split_out:
            o_r[0] = jnp.concatenate([re[:, :dh], ro[:, :dh]], axis=1)
            o_r[1] = jnp.concatenate([re[:, dh:], ro[:, dh:]], axis=1)
        else:
            o_r[...] = jnp.stack([re, ro], axis=1).reshape(2 * blk2, d_out)

    full = lambda w: pl.BlockSpec(w.shape, lambda i: (0, 0))
    if split_out:
        out_spec = pl.BlockSpec((NC, blk2, 128), lambda i: (0, i, 0))
        out_shape = jax.ShapeDtypeStruct((NC, n_out_pairs, 128), jnp.float32)
    else:
        out_spec = pl.BlockSpec((2 * blk2, d_out), lambda i: (i, 0))
        out_shape = jax.ShapeDtypeStruct((2 * n_out_pairs, d_out), jnp.float32)
    return pl.pallas_call(
        body,
        grid=(n_out_pairs // blk2,),
        in_specs=[
            pl.BlockSpec((NC, blk2, 128), lambda i: (0, i, 0)),
            full(Ws[0]), full(bs[0]), full(Ws[1]), full(bs[1]),
            full(Ws[2]), full(bs[2]),
        ],
        out_specs=out_spec,
        out_shape=out_shape,
    )(a_pairs, Ws[0], bs[0], Ws[1], bs[1], Ws[2], bs[2])


def kernel(x, edge_index, params):
    n, d = x.shape
    dh = d // NC
    e = edge_index.shape[1]
    n_rows = ((n + NS * K - 1) // (NS * K)) * NS * K
    n_chunks = -(-e // K)

    ei = edge_index.astype(jnp.int32)
    if e % K:
        pad = n_chunks * K - e
        pad_src = jnp.arange(pad, dtype=jnp.int32) % n
        pad_dst = n + jnp.arange(pad, dtype=jnp.int32) % (n_rows - n)
        ei = jnp.concatenate([ei, jnp.stack([pad_src, pad_dst])], axis=1)
    ei = ei.reshape(2, n_chunks, K)

    x_pad = jnp.zeros((n_rows, d), jnp.float32).at[:n].set(x)
    h = x_pad.reshape(n_rows, NC, dh).transpose(1, 0, 2)

    agg = _make_agg_kernel(n_rows, dh, n_chunks)

    for li, (Ws, bs) in enumerate(params):
        a = agg(h, ei)
        a_pairs = a.reshape(NC, n_rows // 2, d)
        bs2 = tuple(b.reshape(1, -1) for b in bs)
        last = li == len(params) - 1
        if last:
            h = _mlp_call(a_pairs, Ws, bs2, n // 2 // 5, n // 2,
                          split_out=False)
        else:
            hp = _mlp_call(a_pairs, Ws, bs2, 1024, n_rows // 2,
                           split_out=True)
            h = hp.reshape(NC, n_rows, dh)
    return h
